# baseline (speedup 1.0000x reference)
import functools

import jax
import jax.numpy as jnp
from jax import lax
from jax.experimental import pallas as pl
from jax.experimental.pallas import tpu as pltpu
from jax.experimental.pallas import tpu_sc as plsc

NC = 2
NS = 16
LANES = 16


def _sc_coeff_kernel(nnz, n_acc):
    cnt_a = nnz // NS
    cnt_b = nnz // (NC * NS)
    naq = 4
    aq = cnt_a // naq
    zslice = n_acc // NS
    mesh = plsc.VectorSubcoreMesh(core_axis_name="c", subcore_axis_name="s")

    @functools.partial(
        pl.kernel,
        out_type=jax.ShapeDtypeStruct((NC, n_acc), jnp.float32),
        mesh=mesh,
        scratch_types=[
            pltpu.VMEM((cnt_a,), jnp.float32),
            pltpu.VMEM((cnt_a,), jnp.int32),
            pltpu.VMEM((cnt_b,), jnp.int32),
            pltpu.VMEM((cnt_b,), jnp.float32),
            pltpu.VMEM((cnt_b,), jnp.float32),
            pltpu.VMEM((n_acc,), jnp.float32),
            pltpu.VMEM_SHARED((n_acc,), jnp.float32),
            pltpu.VMEM_SHARED((n_acc,), jnp.float32),
        ] + [pltpu.SemaphoreType.DMA] * 17,
    )
    def sc_kernel(seg_hbm, val_hbm, idx_hbm, out_hbm,
                  v_ref, s_ref, i_ref, w_ref, g_ref, dloc,
                  denom_sp, c_sp, sem_v, sem_s, sem_i, sem_g, sem_w,
                  *sem_a):
        cid = lax.axis_index("c")
        sid = lax.axis_index("s")
        lbase = cid * cnt_b
        gbase = sid * cnt_a + cid * cnt_b
        abase = sid * cnt_a

        ld_v = [pltpu.async_copy(val_hbm.at[pl.ds(abase + q * aq, aq)],
                                 v_ref.at[pl.ds(q * aq, aq)], sem_a[q])
                for q in range(naq)]
        ld_s = [pltpu.async_copy(seg_hbm.at[pl.ds(abase + q * aq, aq)],
                                 s_ref.at[pl.ds(q * aq, aq)], sem_a[naq + q])
                for q in range(naq)]
        ld_i = pltpu.async_copy(idx_hbm.at[pl.ds(gbase, cnt_b)], i_ref, sem_i)

        def zb(i, carry):
            dloc[pl.ds(i * LANES, LANES)] = jnp.zeros((LANES,), jnp.float32)
            return carry
        lax.fori_loop(0, zslice // LANES, zb, 0)
        pltpu.sync_copy(dloc.at[pl.ds(0, zslice)], denom_sp.at[pl.ds(sid * zslice, zslice)])
        pltpu.sync_copy(dloc.at[pl.ds(0, zslice)], c_sp.at[pl.ds(sid * zslice, zslice)])

        plsc.subcore_barrier()

        scs = []
        for q in range(naq):
            ld_v[q].wait()
            ld_s[q].wait()
            scs.append(pltpu.async_copy(
                v_ref.at[pl.ds(q * aq, aq)],
                denom_sp.at[s_ref.at[pl.ds(q * aq, aq)]], sem_a[2 * naq + q],
                add=True))
        for dsc in scs:
            dsc.wait()

        plsc.subcore_barrier()

        pltpu.sync_copy(denom_sp.at[pl.ds(sid * zslice, zslice)], dloc.at[pl.ds(0, zslice)])

        def rrow(t, carry):
            sl = pl.ds(t * LANES, LANES)
            dloc[sl] = 1.0 / dloc[sl]
            return carry
        lax.fori_loop(0, zslice // LANES, rrow, 0)
        pltpu.sync_copy(dloc.at[pl.ds(0, zslice)], denom_sp.at[pl.ds(sid * zslice, zslice)])

        plsc.subcore_barrier()

        ld_i.wait()
        nch = 5
        csz = cnt_b // nch
        scats = []
        gat = pltpu.async_copy(
            denom_sp.at[s_ref.at[pl.ds(lbase, csz)]],
            g_ref.at[pl.ds(0, csz)], sem_g)
        for ch in range(nch):
            gat.wait()
            if ch + 1 < nch:
                gat = pltpu.async_copy(
                    denom_sp.at[s_ref.at[pl.ds(lbase + (ch + 1) * csz, csz)]],
                    g_ref.at[pl.ds((ch + 1) * csz, csz)], sem_g)
            base = ch * csz

            def wrow(t, carry, base=base):
                sl = pl.ds(base + t * LANES, LANES)
                w_ref[sl] = v_ref[pl.ds(lbase + base + t * LANES, LANES)] * g_ref[sl]
                return carry
            lax.fori_loop(0, csz // LANES, wrow, 0)
            scats.append(pltpu.async_copy(
                w_ref.at[pl.ds(base, csz)],
                c_sp.at[i_ref.at[pl.ds(base, csz)]], sem_w, add=True))
        for dsc in scats:
            dsc.wait()

        plsc.subcore_barrier()

        @pl.when(sid == 0)
        def _out():
            pltpu.sync_copy(c_sp, dloc)
            pltpu.sync_copy(dloc, out_hbm.at[cid])

    return sc_kernel


def _tc_matvec(n, l_ref, c_ref, f_ref, out_ref):
    cc = c_ref[0:1, 0:n] + c_ref[1:2, 0:n]
    acc = jnp.dot(cc, f_ref[...], preferred_element_type=jnp.float32)
    out_ref[...] = acc * (1.0 / l_ref[0, 0].astype(jnp.float32))


def kernel(feature, indices, values, segment_ids, index_length):
    n, d = feature.shape
    nnz = indices.shape[0]
    n_acc = ((n + NS * LANES - 1) // (NS * LANES)) * (NS * LANES)

    c_part = _sc_coeff_kernel(nnz, n_acc)(segment_ids, values, indices)

    lraw = jnp.reshape(jnp.asarray(index_length, jnp.int32), (1, 1))
    out = pl.pallas_call(
        functools.partial(_tc_matvec, n),
        out_shape=jax.ShapeDtypeStruct((1, d), jnp.float32),
        in_specs=[
            pl.BlockSpec(memory_space=pltpu.SMEM),
            pl.BlockSpec(memory_space=pltpu.VMEM),
            pl.BlockSpec(memory_space=pltpu.VMEM),
        ],
    )(lraw, c_part, feature)

    return out.reshape(d)


# speedup vs baseline: 1.0101x; 1.0101x over previous
import functools

import jax
import jax.numpy as jnp
from jax import lax
from jax.experimental import pallas as pl
from jax.experimental.pallas import tpu as pltpu
from jax.experimental.pallas import tpu_sc as plsc

NC = 2
NS = 16
LANES = 16


def _sc_coeff_kernel(nnz, n_acc):
    cnt_a = nnz // NS
    cnt_b = nnz // (NC * NS)
    naq = 2
    aq = cnt_a // naq
    zslice = n_acc // NS
    mesh = plsc.VectorSubcoreMesh(core_axis_name="c", subcore_axis_name="s")

    @functools.partial(
        pl.kernel,
        out_type=jax.ShapeDtypeStruct((NC, n_acc), jnp.float32),
        mesh=mesh,
        scratch_types=[
            pltpu.VMEM((cnt_a,), jnp.float32),
            pltpu.VMEM((cnt_a,), jnp.int32),
            pltpu.VMEM((cnt_b,), jnp.int32),
            pltpu.VMEM((cnt_b,), jnp.float32),
            pltpu.VMEM((cnt_b,), jnp.float32),
            pltpu.VMEM((n_acc,), jnp.float32),
            pltpu.VMEM_SHARED((n_acc,), jnp.float32),
            pltpu.VMEM_SHARED((n_acc,), jnp.float32),
        ] + [pltpu.SemaphoreType.DMA] * 17,
    )
    def sc_kernel(seg_hbm, val_hbm, idx_hbm, out_hbm,
                  v_ref, s_ref, i_ref, w_ref, g_ref, dloc,
                  denom_sp, c_sp, sem_v, sem_s, sem_i, sem_g, sem_w,
                  *sem_a):
        cid = lax.axis_index("c")
        sid = lax.axis_index("s")
        lbase = cid * cnt_b
        gbase = sid * cnt_a + cid * cnt_b
        abase = sid * cnt_a

        ld_v = [pltpu.async_copy(val_hbm.at[pl.ds(abase + q * aq, aq)],
                                 v_ref.at[pl.ds(q * aq, aq)], sem_a[q])
                for q in range(naq)]
        ld_s = [pltpu.async_copy(seg_hbm.at[pl.ds(abase + q * aq, aq)],
                                 s_ref.at[pl.ds(q * aq, aq)], sem_a[naq + q])
                for q in range(naq)]
        ld_i = pltpu.async_copy(idx_hbm.at[pl.ds(gbase, cnt_b)], i_ref, sem_i)

        def zb(i, carry):
            dloc[pl.ds(i * LANES, LANES)] = jnp.zeros((LANES,), jnp.float32)
            return carry
        lax.fori_loop(0, zslice // LANES, zb, 0)
        pltpu.sync_copy(dloc.at[pl.ds(0, zslice)], denom_sp.at[pl.ds(sid * zslice, zslice)])
        pltpu.sync_copy(dloc.at[pl.ds(0, zslice)], c_sp.at[pl.ds(sid * zslice, zslice)])

        plsc.subcore_barrier()

        scs = []
        for q in range(naq):
            ld_v[q].wait()
            ld_s[q].wait()
            scs.append(pltpu.async_copy(
                v_ref.at[pl.ds(q * aq, aq)],
                denom_sp.at[s_ref.at[pl.ds(q * aq, aq)]], sem_a[2 * naq + q],
                add=True))
        for dsc in scs:
            dsc.wait()

        plsc.subcore_barrier()

        pltpu.sync_copy(denom_sp.at[pl.ds(sid * zslice, zslice)], dloc.at[pl.ds(0, zslice)])

        def rrow(t, carry):
            sl = pl.ds(t * LANES, LANES)
            dloc[sl] = 1.0 / dloc[sl]
            return carry
        lax.fori_loop(0, zslice // LANES, rrow, 0)
        pltpu.sync_copy(dloc.at[pl.ds(0, zslice)], denom_sp.at[pl.ds(sid * zslice, zslice)])

        plsc.subcore_barrier()

        ld_i.wait()
        nch = 5
        csz = cnt_b // nch
        scats = []
        gat = pltpu.async_copy(
            denom_sp.at[s_ref.at[pl.ds(lbase, csz)]],
            g_ref.at[pl.ds(0, csz)], sem_g)
        for ch in range(nch):
            gat.wait()
            if ch + 1 < nch:
                gat = pltpu.async_copy(
                    denom_sp.at[s_ref.at[pl.ds(lbase + (ch + 1) * csz, csz)]],
                    g_ref.at[pl.ds((ch + 1) * csz, csz)], sem_g)
            base = ch * csz

            def wrow(t, carry, base=base):
                sl = pl.ds(base + t * LANES, LANES)
                w_ref[sl] = v_ref[pl.ds(lbase + base + t * LANES, LANES)] * g_ref[sl]
                return carry
            lax.fori_loop(0, csz // LANES, wrow, 0)
            scats.append(pltpu.async_copy(
                w_ref.at[pl.ds(base, csz)],
                c_sp.at[i_ref.at[pl.ds(base, csz)]], sem_w, add=True))
        for dsc in scats:
            dsc.wait()

        plsc.subcore_barrier()

        @pl.when(sid == 0)
        def _out():
            pltpu.sync_copy(c_sp, dloc)
            pltpu.sync_copy(dloc, out_hbm.at[cid])

    return sc_kernel


def _tc_matvec(n, l_ref, c_ref, f_ref, out_ref):
    cc = c_ref[0:1, 0:n] + c_ref[1:2, 0:n]
    acc = jnp.dot(cc, f_ref[...], preferred_element_type=jnp.float32)
    out_ref[...] = acc * (1.0 / l_ref[0, 0].astype(jnp.float32))


def kernel(feature, indices, values, segment_ids, index_length):
    n, d = feature.shape
    nnz = indices.shape[0]
    n_acc = ((n + NS * LANES - 1) // (NS * LANES)) * (NS * LANES)

    c_part = _sc_coeff_kernel(nnz, n_acc)(segment_ids, values, indices)

    lraw = jnp.reshape(jnp.asarray(index_length, jnp.int32), (1, 1))
    out = pl.pallas_call(
        functools.partial(_tc_matvec, n),
        out_shape=jax.ShapeDtypeStruct((1, d), jnp.float32),
        in_specs=[
            pl.BlockSpec(memory_space=pltpu.SMEM),
            pl.BlockSpec(memory_space=pltpu.VMEM),
            pl.BlockSpec(memory_space=pltpu.VMEM),
        ],
    )(lraw, c_part, feature)

    return out.reshape(d)
